# baseline (speedup 1.0000x reference)
import jax
import jax.numpy as jnp
from jax.experimental import pallas as pl
from jax.experimental.pallas import tpu as pltpu

_N_USERS = 100000
_N_ITEMS = 1000000
_EMB = 64

_N_SLICES = 8


def _slices(n, k):
    bounds = [min(((n * i // k) + 7) // 8 * 8, n) for i in range(k + 1)]
    bounds[0], bounds[-1] = 0, n
    return [(s, e - s) for s, e in zip(bounds[:-1], bounds[1:]) if e > s]


_U_SL = _slices(_N_USERS, _N_SLICES)
_I_SL = _slices(_N_ITEMS, _N_SLICES)


def _body(u_ref, i_ref, o_ref, sem_u, sem_i):
    copies = []
    for start, rows in _U_SL:
        copies.append(
            pltpu.make_async_copy(
                u_ref.at[pl.ds(start, rows)], o_ref.at[pl.ds(start, rows)], sem_u
            )
        )
    for start, rows in _I_SL:
        copies.append(
            pltpu.make_async_copy(
                i_ref.at[pl.ds(start, rows)],
                o_ref.at[pl.ds(_N_USERS + start, rows)],
                sem_i,
            )
        )
    for c in copies:
        c.start()
    for c in copies:
        c.wait()


def kernel(user_emb, item_emb):
    return pl.pallas_call(
        _body,
        out_shape=jax.ShapeDtypeStruct((_N_USERS + _N_ITEMS, _EMB), jnp.float32),
        in_specs=[
            pl.BlockSpec(memory_space=pl.ANY),
            pl.BlockSpec(memory_space=pl.ANY),
        ],
        out_specs=pl.BlockSpec(memory_space=pl.ANY),
        scratch_shapes=[pltpu.SemaphoreType.DMA, pltpu.SemaphoreType.DMA],
    )(user_emb, item_emb)


# speedup vs baseline: 13.7057x; 13.7057x over previous
import jax
import jax.numpy as jnp
from jax.experimental import pallas as pl
from jax.experimental.pallas import tpu as pltpu

_N_USERS = 100000
_N_ITEMS = 1000000
_EMB = 64
_B = 2000
_UB = _N_USERS // _B
_IB = _N_ITEMS // _B


def _body(u_ref, i_ref, o_ref):
    pid = pl.program_id(0)

    @pl.when(pid < _UB)
    def _():
        o_ref[...] = u_ref[...]

    @pl.when(pid >= _UB)
    def _():
        o_ref[...] = i_ref[...]


def kernel(user_emb, item_emb):
    return pl.pallas_call(
        _body,
        grid=(_UB + _IB,),
        out_shape=jax.ShapeDtypeStruct((_N_USERS + _N_ITEMS, _EMB), jnp.float32),
        in_specs=[
            pl.BlockSpec((_B, _EMB), lambda i: (jnp.minimum(i, _UB - 1), 0)),
            pl.BlockSpec((_B, _EMB), lambda i: (jnp.maximum(i - _UB, 0), 0)),
        ],
        out_specs=pl.BlockSpec((_B, _EMB), lambda i: (i, 0)),
    )(user_emb, item_emb)


# speedup vs baseline: 16.1368x; 1.1774x over previous
import jax
import jax.numpy as jnp
from jax.experimental import pallas as pl
from jax.experimental.pallas import tpu as pltpu

_N_USERS = 100000
_N_ITEMS = 1000000
_EMB = 64
_B = 10000
_UB = _N_USERS // _B
_IB = _N_ITEMS // _B


def _body(u_ref, i_ref, o_ref):
    pid = pl.program_id(0)

    @pl.when(pid < _UB)
    def _():
        o_ref[...] = u_ref[...]

    @pl.when(pid >= _UB)
    def _():
        o_ref[...] = i_ref[...]


def kernel(user_emb, item_emb):
    return pl.pallas_call(
        _body,
        grid=(_UB + _IB,),
        out_shape=jax.ShapeDtypeStruct((_N_USERS + _N_ITEMS, _EMB), jnp.float32),
        in_specs=[
            pl.BlockSpec((_B, _EMB), lambda i: (jnp.minimum(i, _UB - 1), 0)),
            pl.BlockSpec((_B, _EMB), lambda i: (jnp.maximum(i - _UB, 0), 0)),
        ],
        out_specs=pl.BlockSpec((_B, _EMB), lambda i: (i, 0)),
    )(user_emb, item_emb)
